# baseline (speedup 1.0000x reference)
import functools

import jax
import jax.numpy as jnp
from jax import lax
from jax.experimental import pallas as pl
from jax.experimental.pallas import tpu as pltpu
from jax.experimental.pallas import tpu_sc as plsc

_D = 32
_CHUNK = 1600
_NW = 32


@functools.partial(jax.jit, static_argnums=(4, 5))
def _gather_weight(table, idx_t, w_t, perm, n_l, n_b):
    bpw = n_b // _NW
    rows_per_w = n_l * bpw
    n_chunks = rows_per_w // _CHUNK
    mesh = plsc.VectorSubcoreMesh(core_axis_name="c", subcore_axis_name="s")

    @functools.partial(
        pl.kernel,
        mesh=mesh,
        out_type=jax.ShapeDtypeStruct((n_l * n_b, _D), jnp.float32),
        compiler_params=pltpu.CompilerParams(use_tc_tiling_on_sc=False),
        scratch_types=[
            pltpu.VMEM((rows_per_w,), jnp.int32),
            pltpu.VMEM((rows_per_w,), jnp.int32),
            pltpu.VMEM((rows_per_w,), jnp.float32),
            pltpu.VMEM((_CHUNK, _D), jnp.float32),
            pltpu.SemaphoreType.DMA,
        ],
    )
    def k(table_hbm, idx_hbm, w_hbm, perm_hbm, out_hbm,
          perm_v, idx_v, wf_v, rows_v, sem):
        wid = lax.axis_index("s") * 2 + lax.axis_index("c")
        b0 = wid * bpw
        base_w = wid * rows_per_w

        pltpu.sync_copy(perm_hbm, perm_v)

        def off_body(i, c):
            perm_v[pl.ds(i * 16, 16)] = perm_v[pl.ds(i * 16, 16)] + b0
            return c

        lax.fori_loop(0, rows_per_w // 16, off_body, 0)

        pltpu.async_copy(idx_hbm.at[perm_v], idx_v, sem).wait()
        pltpu.async_copy(w_hbm.at[perm_v], wf_v, sem).wait()

        def chunk_body(g, carry):
            pltpu.async_copy(
                table_hbm.at[idx_v.at[pl.ds(g * _CHUNK, _CHUNK)]], rows_v, sem
            ).wait()

            def group_body(g16, c):
                base16 = g16 * 16
                wvec = wf_v[pl.ds(g * _CHUNK + base16, 16)]
                for j in range(16):
                    wb = lax.broadcast(wvec[j], (16,))
                    i = base16 + j
                    rows_v[i, 0:16] = rows_v[i, 0:16] * wb
                    rows_v[i, 16:32] = rows_v[i, 16:32] * wb
                return c

            lax.fori_loop(0, _CHUNK // 16, group_body, 0)
            pltpu.sync_copy(rows_v, out_hbm.at[pl.ds(base_w + g * _CHUNK, _CHUNK)])
            return carry

        lax.fori_loop(0, n_chunks, chunk_body, 0)

    return k(table, idx_t, w_t, perm)


def kernel(target_indices, target_weights, embedding_weight):
    b, l = target_indices.shape
    idx_t = target_indices.T.astype(jnp.int32).reshape(l * b)
    w_t = target_weights.T.reshape(l * b)
    bpw = b // _NW
    i = jnp.arange(l * bpw, dtype=jnp.int32)
    perm = (i % l) * b + i // l
    out = _gather_weight(embedding_weight, idx_t, w_t, perm, l, b)
    return out.reshape(b, l, _D)


# speedup vs baseline: 1.2336x; 1.2336x over previous
import functools

import jax
import jax.numpy as jnp
from jax import lax
from jax.experimental import pallas as pl
from jax.experimental.pallas import tpu as pltpu
from jax.experimental.pallas import tpu_sc as plsc

_D = 32
_BC = 32
_NW = 32


@functools.partial(jax.jit, static_argnums=(3, 4))
def _gather_weight(table, idx, w, n_b, n_l):
    bpw = n_b // _NW
    n_chunks = bpw // _BC
    chunk_rows = _BC * n_l
    rows_per_w = bpw * n_l
    mesh = plsc.VectorSubcoreMesh(core_axis_name="c", subcore_axis_name="s")

    @functools.partial(
        pl.kernel,
        mesh=mesh,
        out_type=jax.ShapeDtypeStruct((n_b, n_l, _D), jnp.float32),
        compiler_params=pltpu.CompilerParams(use_tc_tiling_on_sc=False),
        scratch_types=[
            pltpu.VMEM((bpw, n_l), jnp.int32),
            pltpu.VMEM((bpw, n_l), jnp.float32),
            pltpu.VMEM((rows_per_w,), jnp.int32),
            pltpu.VMEM((rows_per_w,), jnp.float32),
            pltpu.VMEM((chunk_rows, _D), jnp.float32),
            pltpu.SemaphoreType.DMA,
        ],
    )
    def k(table_hbm, idx_hbm, w_hbm, out_hbm,
          idx2_v, w2_v, idxf_v, wf_v, rows_v, sem):
        wid = lax.axis_index("s") * 2 + lax.axis_index("c")
        b0 = wid * bpw

        pltpu.sync_copy(idx_hbm.at[pl.ds(b0, bpw), :], idx2_v)
        pltpu.sync_copy(w_hbm.at[pl.ds(b0, bpw), :], w2_v)

        starts = (0, 16, 32, n_l - 16)

        def flat_body(b, c):
            base = b * n_l
            for s in starts:
                idxf_v[pl.ds(base + s, 16)] = idx2_v[b, s:s + 16]
                wf_v[pl.ds(base + s, 16)] = w2_v[b, s:s + 16]
            return c

        lax.fori_loop(0, bpw, flat_body, 0)

        def chunk_body(g, carry):
            pltpu.async_copy(
                table_hbm.at[idxf_v.at[pl.ds(g * chunk_rows, chunk_rows)]],
                rows_v, sem,
            ).wait()

            def group_body(g16, c):
                base16 = g16 * 16
                wvec = wf_v[pl.ds(g * chunk_rows + base16, 16)]
                for j in range(16):
                    wb = lax.broadcast(wvec[j], (16,))
                    i = base16 + j
                    rows_v[i, 0:16] = rows_v[i, 0:16] * wb
                    rows_v[i, 16:32] = rows_v[i, 16:32] * wb
                return c

            lax.fori_loop(0, chunk_rows // 16, group_body, 0)

            def out_body(br, c):
                pltpu.sync_copy(
                    rows_v.at[pl.ds(br * n_l, n_l), :],
                    out_hbm.at[b0 + g * _BC + br],
                )
                return c

            lax.fori_loop(0, _BC, out_body, 0)
            return carry

        lax.fori_loop(0, n_chunks, chunk_body, 0)

    return k(table, idx, w)


def kernel(target_indices, target_weights, embedding_weight):
    b, l = target_indices.shape
    return _gather_weight(
        embedding_weight, target_indices.astype(jnp.int32), target_weights, b, l
    )


# speedup vs baseline: 1.2453x; 1.0095x over previous
import functools

import jax
import jax.numpy as jnp
from jax import lax
from jax.experimental import pallas as pl
from jax.experimental.pallas import tpu as pltpu
from jax.experimental.pallas import tpu_sc as plsc

_D = 32
_LC = 10
_NW = 32


@functools.partial(jax.jit, static_argnums=(3, 4))
def _gather_weight(table, idx_t, w_t, n_b, n_l):
    bpw = n_b // _NW
    n_chunks = n_l // _LC
    chunk_rows = _LC * bpw
    rows_per_w = bpw * n_l
    mesh = plsc.VectorSubcoreMesh(core_axis_name="c", subcore_axis_name="s")

    @functools.partial(
        pl.kernel,
        mesh=mesh,
        out_type=jax.ShapeDtypeStruct((n_b, n_l, _D), jnp.float32),
        compiler_params=pltpu.CompilerParams(use_tc_tiling_on_sc=False),
        scratch_types=[
            pltpu.VMEM((n_l, bpw), jnp.int32),
            pltpu.VMEM((n_l, bpw), jnp.float32),
            pltpu.VMEM((rows_per_w,), jnp.int32),
            pltpu.VMEM((rows_per_w,), jnp.float32),
            pltpu.VMEM((chunk_rows, _D), jnp.float32),
            pltpu.SemaphoreType.DMA,
        ],
    )
    def k(table_hbm, idx_hbm, w_hbm, out_hbm,
          idx2_v, w2_v, idxf_v, wf_v, rows_v, sem):
        wid = lax.axis_index("s") * 2 + lax.axis_index("c")
        b0 = wid * bpw

        pltpu.sync_copy(idx_hbm.at[:, pl.ds(b0, bpw)], idx2_v)
        pltpu.sync_copy(w_hbm.at[:, pl.ds(b0, bpw)], w2_v)

        def flat_body(l, c):
            base = l * bpw
            for s in range(0, bpw, 16):
                idxf_v[pl.ds(base + s, 16)] = idx2_v[l, s:s + 16]
                wf_v[pl.ds(base + s, 16)] = w2_v[l, s:s + 16]
            return c

        lax.fori_loop(0, n_l, flat_body, 0)

        def chunk_body(g, carry):
            pltpu.async_copy(
                table_hbm.at[idxf_v.at[pl.ds(g * chunk_rows, chunk_rows)]],
                rows_v, sem,
            ).wait()

            def group_body(g16, c):
                base16 = g16 * 16
                wvec = wf_v[pl.ds(g * chunk_rows + base16, 16)]
                for j in range(16):
                    wb = lax.broadcast(wvec[j], (16,))
                    i = base16 + j
                    rows_v[i, 0:16] = rows_v[i, 0:16] * wb
                    rows_v[i, 16:32] = rows_v[i, 16:32] * wb
                return c

            lax.fori_loop(0, chunk_rows // 16, group_body, 0)

            def out_body(lr, c):
                pltpu.sync_copy(
                    rows_v.at[pl.ds(lr * bpw, bpw), :],
                    out_hbm.at[pl.ds(b0, bpw), g * _LC + lr, :],
                )
                return c

            lax.fori_loop(0, _LC, out_body, 0)
            return carry

        lax.fori_loop(0, n_chunks, chunk_body, 0)

    return k(table, idx_t, w_t)


def kernel(target_indices, target_weights, embedding_weight):
    b, l = target_indices.shape
    return _gather_weight(
        embedding_weight,
        target_indices.T.astype(jnp.int32),
        target_weights.T,
        b, l,
    )
